# baseline (speedup 1.0000x reference)
import functools

import jax
import jax.numpy as jnp
from jax import lax
from jax.experimental import pallas as pl
from jax.experimental.pallas import tpu as pltpu
from jax.experimental.pallas import tpu_sc as plsc

B, T, CIN, N = 8, 32, 132, 64
HID, OFF, TOPK = 256, 4, 16
BN = B * N
KN = TOPK * N
FAN = CIN + OFF + HID
NPAIR = T * B
NWORK = 32
PPW = NPAIR // NWORK


def _knn_sc_body(cent_hbm, past_hbm, sel_hbm, cbuf, pbuf, selbuf):
    wid = lax.axis_index("s") * 2 + lax.axis_index("c")
    iotav = lax.iota(jnp.int32, 16)

    def merge(ak, av, bk, bv):
        rbk = lax.rev(bk, (0,))
        rbv = lax.rev(bv, (0,))
        m = ak <= rbk
        lk = jnp.where(m, ak, rbk)
        lv = jnp.where(m, av, rbv)
        return plsc.sort_key_val(lk, lv)

    def pair_body(i, carry):
        pair = wid * PPW + i
        pltpu.sync_copy(cent_hbm.at[pair], cbuf)
        pltpu.sync_copy(past_hbm.at[pair], pbuf)
        pvt = [[pbuf[pl.ds(c * N + j * 16, 16)] for j in range(4)]
               for c in range(OFF)]

        def g_body(g, gcarry):
            cvecs = [cbuf[pl.ds(c * N + g * 16, 16)] for c in range(OFF)]
            for q in range(16):
                runs = []
                for j in range(4):
                    acc = None
                    for c in range(OFF):
                        diff = cvecs[c][q] - pvt[c][j]
                        sq = diff * diff
                        acc = sq if acc is None else acc + sq
                    runs.append(plsc.sort_key_val(acc, iotav + j * 16))
                k0, v0 = merge(*runs[0], *runs[1])
                k1, v1 = merge(*runs[2], *runs[3])
                _, fv = merge(k0, v0, k1, v1)
                plsc.store_scatter(selbuf, [iotav * N + (g * 16 + q)], fv)
            return gcarry

        lax.fori_loop(0, 4, g_body, 0)
        pltpu.sync_copy(selbuf, sel_hbm.at[pair])
        return carry

    lax.fori_loop(0, PPW, pair_body, 0)


@functools.cache
def _make_knn_sc():
    return functools.partial(
        pl.kernel,
        out_type=jax.ShapeDtypeStruct((NPAIR, KN), jnp.int32),
        mesh=plsc.VectorSubcoreMesh(
            core_axis_name="c", subcore_axis_name="s", num_cores=2),
        scratch_types=[
            pltpu.VMEM((OFF * N,), jnp.float32),
            pltpu.VMEM((OFF * N,), jnp.float32),
            pltpu.VMEM((KN,), jnp.int32),
        ],
        compiler_params=pltpu.CompilerParams(needs_layout_passes=False),
    )(_knn_sc_body)


def _step_kernel(xs_ref, sel_ref, W_ref, b_ref, out_ref, H, C):
    t = pl.program_id(0)

    @pl.when(t == 0)
    def _():
        H[...] = jnp.zeros_like(H)
        C[...] = jnp.zeros_like(C)

    Wx = W_ref[:, :CIN]
    Wp = W_ref[:, CIN:CIN + OFF]
    Wh = W_ref[:, CIN:]
    Hh = jnp.dot(Wh, H[...], preferred_element_type=jnp.float32)

    iota_g = lax.broadcasted_iota(jnp.int32, (N, KN), 0)
    E = (iota_g == lax.broadcasted_iota(jnp.int32, (N, KN), 1) % N
         ).astype(jnp.float32)
    zeroN = jnp.zeros((HID, N), dtype=jnp.float32)
    for bb in range(B):
        cols = slice(bb * N, (bb + 1) * N)
        xb = xs_ref[bb, 0]
        pos_b = xb[:OFF]
        Ab = (jnp.dot(Wx, xb, preferred_element_type=jnp.float32)
              - jnp.dot(Wp, pos_b, preferred_element_type=jnp.float32)
              + b_ref[...])
        Gb = (iota_g == sel_ref[bb, 0, 0][None, :]).astype(jnp.float32)
        lhs = jnp.concatenate(
            [jnp.concatenate([Hh[:, cols], Ab], axis=1),
             jnp.concatenate([C[:, cols], zeroN], axis=1)], axis=0)
        rhs = jnp.concatenate([Gb, E], axis=0)
        zz = jnp.dot(lhs, rhs, preferred_element_type=jnp.float32)
        zi = zz[0:HID]
        zf = zz[HID:2 * HID]
        zo = zz[2 * HID:3 * HID]
        zg = zz[3 * HID:4 * HID]
        Cgb = zz[4 * HID:]
        cn = (jax.nn.sigmoid(zf) * Cgb
              + jax.nn.sigmoid(zi) * jnp.tanh(zg))
        hn = jax.nn.sigmoid(zo) * jnp.tanh(cn)
        w = KN
        while w > N:
            w //= 2
            cn = jnp.maximum(cn[:, :w], cn[:, w:2 * w])
            hn = jnp.maximum(hn[:, :w], hn[:, w:2 * w])
        C[:, cols] = cn
        H[OFF:, cols] = hn
        H[:OFF, cols] = pos_b
        out_ref[bb, 0, OFF:, :] = hn
        out_ref[bb, 0, :OFF, :] = pos_b


@jax.jit
def kernel(inputs, offsets, W, b):
    pos4 = inputs[:, :, :OFF]
    cent = pos4 - offsets
    past = jnp.concatenate([pos4[:, :1], pos4[:, :-1]], axis=1)
    sel = _make_knn_sc()(cent.reshape(NPAIR, OFF * N),
                         past.reshape(NPAIR, OFF * N))

    b2 = b.reshape(4 * HID, 1)
    sel3 = sel.reshape(B, T, 1, KN)

    out = pl.pallas_call(
        _step_kernel,
        grid=(T,),
        in_specs=[
            pl.BlockSpec((B, 1, CIN, N), lambda t: (0, t, 0, 0)),
            pl.BlockSpec((B, 1, 1, KN), lambda t: (0, t, 0, 0)),
            pl.BlockSpec((4 * HID, FAN), lambda t: (0, 0)),
            pl.BlockSpec((4 * HID, 1), lambda t: (0, 0)),
        ],
        out_specs=pl.BlockSpec((B, 1, OFF + HID, N), lambda t: (0, t, 0, 0)),
        out_shape=jax.ShapeDtypeStruct((B, T, OFF + HID, N), jnp.float32),
        scratch_shapes=[
            pltpu.VMEM((OFF + HID, BN), jnp.float32),
            pltpu.VMEM((HID, BN), jnp.float32),
        ],
        compiler_params=pltpu.CompilerParams(
            dimension_semantics=("arbitrary",),
        ),
    )(inputs, sel3, W, b2)

    ind = jnp.transpose(sel.reshape(B, T, TOPK, N), (0, 1, 3, 2))
    return out, ind


# speedup vs baseline: 1.0129x; 1.0129x over previous
import functools

import jax
import jax.numpy as jnp
from jax import lax
from jax.experimental import pallas as pl
from jax.experimental.pallas import tpu as pltpu
from jax.experimental.pallas import tpu_sc as plsc

B, T, CIN, N = 8, 32, 132, 64
HID, OFF, TOPK = 256, 4, 16
BN = B * N
KN = TOPK * N
FAN = CIN + OFF + HID
NPAIR = T * B
NWORK = 32
PPW = NPAIR // NWORK


_PAIRW = 2 * OFF * N


def _knn_sc_body(cp_hbm, sel_hbm, inbuf, selbuf):
    wid = lax.axis_index("s") * 2 + lax.axis_index("c")
    iotav = lax.iota(jnp.int32, 16)

    pltpu.sync_copy(cp_hbm.at[pl.ds(wid * PPW * _PAIRW, PPW * _PAIRW)], inbuf)

    def merge(ak, av, bk, bv):
        rbk = lax.rev(bk, (0,))
        rbv = lax.rev(bv, (0,))
        m = ak <= rbk
        lk = jnp.where(m, ak, rbk)
        lv = jnp.where(m, av, rbv)
        return plsc.sort_key_val(lk, lv)

    def pair_body(i, carry):
        base = i * _PAIRW
        pvt = [[inbuf[pl.ds(base + OFF * N + c * N + j * 16, 16)]
                for j in range(4)] for c in range(OFF)]

        def g_body(g, gcarry):
            cvecs = [inbuf[pl.ds(base + c * N + g * 16, 16)]
                     for c in range(OFF)]
            ivec = jnp.full((16,), i, dtype=jnp.int32)
            for q in range(16):
                runs = []
                for j in range(4):
                    acc = None
                    for c in range(OFF):
                        diff = cvecs[c][q] - pvt[c][j]
                        sq = diff * diff
                        acc = sq if acc is None else acc + sq
                    runs.append(plsc.sort_key_val(acc, iotav + j * 16))
                k0, v0 = merge(*runs[0], *runs[1])
                k1, v1 = merge(*runs[2], *runs[3])
                _, fv = merge(k0, v0, k1, v1)
                plsc.store_scatter(
                    selbuf, [ivec, iotav * N + (g * 16 + q)], fv)
            return gcarry

        return lax.fori_loop(0, 4, g_body, carry)

    lax.fori_loop(0, PPW, pair_body, 0)
    pltpu.sync_copy(selbuf, sel_hbm.at[pl.ds(wid * PPW, PPW)])


@functools.cache
def _make_knn_sc():
    return functools.partial(
        pl.kernel,
        out_type=jax.ShapeDtypeStruct((NPAIR, KN), jnp.int32),
        mesh=plsc.VectorSubcoreMesh(
            core_axis_name="c", subcore_axis_name="s", num_cores=2),
        scratch_types=[
            pltpu.VMEM((PPW * _PAIRW,), jnp.float32),
            pltpu.VMEM((PPW, KN), jnp.int32),
        ],
        compiler_params=pltpu.CompilerParams(needs_layout_passes=False),
    )(_knn_sc_body)


def _step_kernel(xs_ref, sel_ref, W_ref, b_ref, out_ref, H, C):
    t = pl.program_id(0)

    @pl.when(t == 0)
    def _():
        H[...] = jnp.zeros_like(H)
        C[...] = jnp.zeros_like(C)

    Wx = W_ref[:, :CIN]
    Wp = W_ref[:, CIN:CIN + OFF]
    Wh = W_ref[:, CIN:]
    Hh = jnp.dot(Wh, H[...], preferred_element_type=jnp.float32)

    iota_g = lax.broadcasted_iota(jnp.int32, (N, KN), 0)
    E = (iota_g == lax.broadcasted_iota(jnp.int32, (N, KN), 1) % N
         ).astype(jnp.float32)
    zeroN = jnp.zeros((HID, N), dtype=jnp.float32)
    for bb in range(B):
        cols = slice(bb * N, (bb + 1) * N)
        xb = xs_ref[bb, 0]
        pos_b = xb[:OFF]
        Ab = (jnp.dot(Wx, xb, preferred_element_type=jnp.float32)
              - jnp.dot(Wp, pos_b, preferred_element_type=jnp.float32)
              + b_ref[...])
        Gb = (iota_g == sel_ref[bb, 0, 0][None, :]).astype(jnp.float32)
        lhs = jnp.concatenate(
            [jnp.concatenate([Hh[:, cols], Ab], axis=1),
             jnp.concatenate([C[:, cols], zeroN], axis=1)], axis=0)
        rhs = jnp.concatenate([Gb, E], axis=0)
        zz = jnp.dot(lhs, rhs, preferred_element_type=jnp.float32)
        zi = zz[0:HID]
        zf = zz[HID:2 * HID]
        zo = zz[2 * HID:3 * HID]
        zg = zz[3 * HID:4 * HID]
        Cgb = zz[4 * HID:]
        cn = (jax.nn.sigmoid(zf) * Cgb
              + jax.nn.sigmoid(zi) * jnp.tanh(zg))
        hn = jax.nn.sigmoid(zo) * jnp.tanh(cn)
        w = KN
        while w > N:
            w //= 2
            cn = jnp.maximum(cn[:, :w], cn[:, w:2 * w])
            hn = jnp.maximum(hn[:, :w], hn[:, w:2 * w])
        C[:, cols] = cn
        H[OFF:, cols] = hn
        H[:OFF, cols] = pos_b
        out_ref[bb, 0, OFF:, :] = hn
        out_ref[bb, 0, :OFF, :] = pos_b


@jax.jit
def kernel(inputs, offsets, W, b):
    pos4 = inputs[:, :, :OFF]
    cent = pos4 - offsets
    past = jnp.concatenate([pos4[:, :1], pos4[:, :-1]], axis=1)
    cp = jnp.concatenate([cent.reshape(NPAIR, OFF * N),
                          past.reshape(NPAIR, OFF * N)], axis=1)
    sel = _make_knn_sc()(cp.reshape(-1))

    b2 = b.reshape(4 * HID, 1)
    sel3 = sel.reshape(B, T, 1, KN)

    out = pl.pallas_call(
        _step_kernel,
        grid=(T,),
        in_specs=[
            pl.BlockSpec((B, 1, CIN, N), lambda t: (0, t, 0, 0)),
            pl.BlockSpec((B, 1, 1, KN), lambda t: (0, t, 0, 0)),
            pl.BlockSpec((4 * HID, FAN), lambda t: (0, 0)),
            pl.BlockSpec((4 * HID, 1), lambda t: (0, 0)),
        ],
        out_specs=pl.BlockSpec((B, 1, OFF + HID, N), lambda t: (0, t, 0, 0)),
        out_shape=jax.ShapeDtypeStruct((B, T, OFF + HID, N), jnp.float32),
        scratch_shapes=[
            pltpu.VMEM((OFF + HID, BN), jnp.float32),
            pltpu.VMEM((HID, BN), jnp.float32),
        ],
        compiler_params=pltpu.CompilerParams(
            dimension_semantics=("arbitrary",),
        ),
    )(inputs, sel3, W, b2)

    ind = jnp.transpose(sel.reshape(B, T, TOPK, N), (0, 1, 3, 2))
    return out, ind


# speedup vs baseline: 1.1212x; 1.1069x over previous
import functools

import jax
import jax.numpy as jnp
from jax import lax
from jax.experimental import pallas as pl
from jax.experimental.pallas import tpu as pltpu
from jax.experimental.pallas import tpu_sc as plsc

B, T, CIN, N = 8, 32, 132, 64
HID, OFF, TOPK = 256, 4, 16
BN = B * N
KN = TOPK * N
FAN = CIN + OFF + HID
NPAIR = T * B
NWORK = 32
PPW = NPAIR // NWORK


_PAIRW = 2 * OFF * N


def _knn_sc_body(cp_hbm, sel_hbm, inbuf, selbuf):
    wid = lax.axis_index("s") * 2 + lax.axis_index("c")
    iotav = lax.iota(jnp.int32, 16)

    pltpu.sync_copy(cp_hbm.at[pl.ds(wid * PPW * _PAIRW, PPW * _PAIRW)], inbuf)

    def merge(ak, av, bk, bv):
        rbk = lax.rev(bk, (0,))
        rbv = lax.rev(bv, (0,))
        m = ak <= rbk
        lk = jnp.where(m, ak, rbk)
        lv = jnp.where(m, av, rbv)
        return plsc.sort_key_val(lk, lv)

    def pair_body(i, carry):
        base = i * _PAIRW
        pvt = [[inbuf[pl.ds(base + OFF * N + c * N + j * 16, 16)]
                for j in range(4)] for c in range(OFF)]

        def g_body(g, gcarry):
            cvecs = [inbuf[pl.ds(base + c * N + g * 16, 16)]
                     for c in range(OFF)]
            ivec = jnp.full((16,), i, dtype=jnp.int32)
            for q in range(16):
                runs = []
                for j in range(4):
                    acc = None
                    for c in range(OFF):
                        diff = cvecs[c][q] - pvt[c][j]
                        sq = diff * diff
                        acc = sq if acc is None else acc + sq
                    runs.append(plsc.sort_key_val(acc, iotav + j * 16))
                k0, v0 = merge(*runs[0], *runs[1])
                k1, v1 = merge(*runs[2], *runs[3])
                _, fv = merge(k0, v0, k1, v1)
                plsc.store_scatter(
                    selbuf, [ivec, iotav * N + (g * 16 + q)], fv)
            return gcarry

        return lax.fori_loop(0, 4, g_body, carry)

    lax.fori_loop(0, PPW, pair_body, 0)
    pltpu.sync_copy(selbuf, sel_hbm.at[pl.ds(wid * PPW, PPW)])


@functools.cache
def _make_knn_sc():
    return functools.partial(
        pl.kernel,
        out_type=jax.ShapeDtypeStruct((NPAIR, KN), jnp.int32),
        mesh=plsc.VectorSubcoreMesh(
            core_axis_name="c", subcore_axis_name="s", num_cores=2),
        scratch_types=[
            pltpu.VMEM((PPW * _PAIRW,), jnp.float32),
            pltpu.VMEM((PPW, KN), jnp.int32),
        ],
        compiler_params=pltpu.CompilerParams(needs_layout_passes=False),
    )(_knn_sc_body)


def _step_kernel(xs_ref, sel_ref, W_ref, b_ref, out_ref, H, C):
    t = pl.program_id(0)

    @pl.when(t == 0)
    def _():
        H[...] = jnp.zeros_like(H)
        C[...] = jnp.zeros_like(C)

    Wx = W_ref[:, :CIN]
    Wp = W_ref[:, CIN:CIN + OFF]
    Wh = W_ref[:, CIN:]
    Hh = jnp.dot(Wh, H[...], preferred_element_type=jnp.float32)

    iota_g = lax.broadcasted_iota(jnp.int32, (N, KN), 0)
    E = (iota_g == lax.broadcasted_iota(jnp.int32, (N, KN), 1) % N
         ).astype(jnp.float32)
    zeroN = jnp.zeros((HID, N), dtype=jnp.float32)
    for bb in range(B):
        cols = slice(bb * N, (bb + 1) * N)
        xb = xs_ref[bb, 0]
        pos_b = xb[:OFF]
        Ab = (jnp.dot(Wx, xb, preferred_element_type=jnp.float32)
              - jnp.dot(Wp, pos_b, preferred_element_type=jnp.float32)
              + b_ref[...])
        Gb = (iota_g == sel_ref[bb, 0, 0][None, :]).astype(jnp.float32)
        lhs = jnp.concatenate(
            [jnp.concatenate([Hh[:, cols], Ab], axis=1),
             jnp.concatenate([C[:, cols], zeroN], axis=1)], axis=0)
        rhs = jnp.concatenate([Gb, E], axis=0)
        zz = jnp.dot(lhs, rhs, preferred_element_type=jnp.float32)
        zi = zz[0:HID]
        zf = zz[HID:2 * HID]
        zo = zz[2 * HID:3 * HID]
        zg = zz[3 * HID:4 * HID]
        Cgb = zz[4 * HID:]

        def sig(x):
            return 0.5 * jnp.tanh(0.5 * x) + 0.5

        cn = sig(zf) * Cgb + sig(zi) * jnp.tanh(zg)
        hn = sig(zo) * jnp.tanh(cn)
        w = KN
        while w > N:
            w //= 2
            cn = jnp.maximum(cn[:, :w], cn[:, w:2 * w])
            hn = jnp.maximum(hn[:, :w], hn[:, w:2 * w])
        C[:, cols] = cn
        H[OFF:, cols] = hn
        H[:OFF, cols] = pos_b
        out_ref[bb, 0, OFF:, :] = hn
        out_ref[bb, 0, :OFF, :] = pos_b


@jax.jit
def kernel(inputs, offsets, W, b):
    pos4 = inputs[:, :, :OFF]
    cent = pos4 - offsets
    past = jnp.concatenate([pos4[:, :1], pos4[:, :-1]], axis=1)
    cp = jnp.concatenate([cent.reshape(NPAIR, OFF * N),
                          past.reshape(NPAIR, OFF * N)], axis=1)
    sel = _make_knn_sc()(cp.reshape(-1))

    b2 = b.reshape(4 * HID, 1)
    sel3 = sel.reshape(B, T, 1, KN)

    out = pl.pallas_call(
        _step_kernel,
        grid=(T,),
        in_specs=[
            pl.BlockSpec((B, 1, CIN, N), lambda t: (0, t, 0, 0)),
            pl.BlockSpec((B, 1, 1, KN), lambda t: (0, t, 0, 0)),
            pl.BlockSpec((4 * HID, FAN), lambda t: (0, 0)),
            pl.BlockSpec((4 * HID, 1), lambda t: (0, 0)),
        ],
        out_specs=pl.BlockSpec((B, 1, OFF + HID, N), lambda t: (0, t, 0, 0)),
        out_shape=jax.ShapeDtypeStruct((B, T, OFF + HID, N), jnp.float32),
        scratch_shapes=[
            pltpu.VMEM((OFF + HID, BN), jnp.float32),
            pltpu.VMEM((HID, BN), jnp.float32),
        ],
        compiler_params=pltpu.CompilerParams(
            dimension_semantics=("arbitrary",),
        ),
    )(inputs, sel3, W, b2)

    ind = jnp.transpose(sel.reshape(B, T, TOPK, N), (0, 1, 3, 2))
    return out, ind


# speedup vs baseline: 1.1767x; 1.0496x over previous
import functools

import jax
import jax.numpy as jnp
from jax import lax
from jax.experimental import pallas as pl
from jax.experimental.pallas import tpu as pltpu
from jax.experimental.pallas import tpu_sc as plsc

B, T, CIN, N = 8, 32, 132, 64
HID, OFF, TOPK = 256, 4, 16
BN = B * N
KN = TOPK * N
FAN = CIN + OFF + HID
NPAIR = T * B
NWORK = 32
PPW = NPAIR // NWORK


_PAIRW = 2 * OFF * N


def _knn_sc_body(cp_hbm, sel_hbm, inbuf, selbuf):
    wid = lax.axis_index("s") * 2 + lax.axis_index("c")
    iotav = lax.iota(jnp.int32, 16)

    pltpu.sync_copy(cp_hbm.at[pl.ds(wid * PPW * _PAIRW, PPW * _PAIRW)], inbuf)

    def merge(ak, av, bk, bv):
        rbk = lax.rev(bk, (0,))
        rbv = lax.rev(bv, (0,))
        m = ak <= rbk
        lk = jnp.where(m, ak, rbk)
        lv = jnp.where(m, av, rbv)
        return plsc.sort_key_val(lk, lv)

    def pair_body(i, carry):
        base = i * _PAIRW
        pvt = [[inbuf[pl.ds(base + OFF * N + c * N + j * 16, 16)]
                for j in range(4)] for c in range(OFF)]

        def g_body(g, gcarry):
            cvecs = [inbuf[pl.ds(base + c * N + g * 16, 16)]
                     for c in range(OFF)]
            ivec = jnp.full((16,), i, dtype=jnp.int32)
            for q in range(16):
                runs = []
                for j in range(4):
                    acc = None
                    for c in range(OFF):
                        diff = cvecs[c][q] - pvt[c][j]
                        sq = diff * diff
                        acc = sq if acc is None else acc + sq
                    runs.append(plsc.sort_key_val(acc, iotav + j * 16))
                k0, v0 = merge(*runs[0], *runs[1])
                k1, v1 = merge(*runs[2], *runs[3])
                _, fv = merge(k0, v0, k1, v1)
                plsc.store_scatter(
                    selbuf, [ivec, iotav * N + (g * 16 + q)], fv)
            return gcarry

        return lax.fori_loop(0, 4, g_body, carry)

    lax.fori_loop(0, PPW, pair_body, 0)
    pltpu.sync_copy(selbuf, sel_hbm.at[pl.ds(wid * PPW, PPW)])


@functools.cache
def _make_knn_sc():
    return functools.partial(
        pl.kernel,
        out_type=jax.ShapeDtypeStruct((NPAIR, KN), jnp.int32),
        mesh=plsc.VectorSubcoreMesh(
            core_axis_name="c", subcore_axis_name="s", num_cores=2),
        scratch_types=[
            pltpu.VMEM((PPW * _PAIRW,), jnp.float32),
            pltpu.VMEM((PPW, KN), jnp.int32),
        ],
        compiler_params=pltpu.CompilerParams(needs_layout_passes=False),
    )(_knn_sc_body)


def _step_kernel(xs_ref, sel_ref, W_ref, b_ref, out_ref, H, C):
    t = pl.program_id(0)

    @pl.when(t == 0)
    def _():
        H[...] = jnp.zeros_like(H)
        C[...] = jnp.zeros_like(C)

    Wx = W_ref[:, :CIN]
    Wp = W_ref[:, CIN:CIN + OFF]
    Wh = W_ref[:, CIN:]
    Hh = jnp.dot(Wh, H[...], preferred_element_type=jnp.float32)

    x_t = xs_ref[0]
    pos_all = x_t[:OFF]
    Afull = (jnp.dot(Wx, x_t, preferred_element_type=jnp.float32)
             - jnp.dot(Wp, pos_all, preferred_element_type=jnp.float32)
             + b_ref[...])

    iota_g = lax.broadcasted_iota(jnp.int32, (N, KN), 0)
    E = (iota_g == lax.broadcasted_iota(jnp.int32, (N, KN), 1) % N
         ).astype(jnp.float32)
    zeroN = jnp.zeros((HID, N), dtype=jnp.float32)
    for bb in range(B):
        cols = slice(bb * N, (bb + 1) * N)
        pos_b = pos_all[:, cols]
        Ab = Afull[:, cols]
        Gb = (iota_g == sel_ref[bb, 0, 0][None, :]).astype(jnp.float32)
        lhs = jnp.concatenate(
            [jnp.concatenate([Hh[:, cols], Ab], axis=1),
             jnp.concatenate([C[:, cols], zeroN], axis=1)], axis=0)
        rhs = jnp.concatenate([Gb, E], axis=0)
        zz = jnp.dot(lhs, rhs, preferred_element_type=jnp.float32)
        zi = zz[0:HID]
        zf = zz[HID:2 * HID]
        zo = zz[2 * HID:3 * HID]
        zg = zz[3 * HID:4 * HID]
        Cgb = zz[4 * HID:]

        def sig(x):
            return 0.5 * jnp.tanh(0.5 * x) + 0.5

        cn = sig(zf) * Cgb + sig(zi) * jnp.tanh(zg)
        hn = sig(zo) * jnp.tanh(cn)
        w = KN
        while w > N:
            w //= 2
            cn = jnp.maximum(cn[:, :w], cn[:, w:2 * w])
            hn = jnp.maximum(hn[:, :w], hn[:, w:2 * w])
        C[:, cols] = cn
        H[OFF:, cols] = hn
        H[:OFF, cols] = pos_b
        out_ref[bb, 0, OFF:, :] = hn
        out_ref[bb, 0, :OFF, :] = pos_b


@jax.jit
def kernel(inputs, offsets, W, b):
    pos4 = inputs[:, :, :OFF]
    cent = pos4 - offsets
    past = jnp.concatenate([pos4[:, :1], pos4[:, :-1]], axis=1)
    cp = jnp.concatenate([cent.reshape(NPAIR, OFF * N),
                          past.reshape(NPAIR, OFF * N)], axis=1)
    sel = _make_knn_sc()(cp.reshape(-1))

    b2 = b.reshape(4 * HID, 1)
    sel3 = sel.reshape(B, T, 1, KN)
    xs_t = jnp.transpose(inputs, (1, 2, 0, 3)).reshape(T, CIN, BN)

    out = pl.pallas_call(
        _step_kernel,
        grid=(T,),
        in_specs=[
            pl.BlockSpec((1, CIN, BN), lambda t: (t, 0, 0)),
            pl.BlockSpec((B, 1, 1, KN), lambda t: (0, t, 0, 0)),
            pl.BlockSpec((4 * HID, FAN), lambda t: (0, 0)),
            pl.BlockSpec((4 * HID, 1), lambda t: (0, 0)),
        ],
        out_specs=pl.BlockSpec((B, 1, OFF + HID, N), lambda t: (0, t, 0, 0)),
        out_shape=jax.ShapeDtypeStruct((B, T, OFF + HID, N), jnp.float32),
        scratch_shapes=[
            pltpu.VMEM((OFF + HID, BN), jnp.float32),
            pltpu.VMEM((HID, BN), jnp.float32),
        ],
        compiler_params=pltpu.CompilerParams(
            dimension_semantics=("arbitrary",),
        ),
    )(xs_t, sel3, W, b2)

    ind = jnp.transpose(sel.reshape(B, T, TOPK, N), (0, 1, 3, 2))
    return out, ind
